# baseline (speedup 1.0000x reference)
import functools

import jax
import jax.numpy as jnp
from jax import lax
from jax.experimental import pallas as pl
from jax.experimental.pallas import tpu as pltpu
from jax.experimental.pallas import tpu_sc as plsc

_N = 10000
_H = 128
_E = 320000
_NC = 2
_NS = 16
_NW = _NC * _NS
_EPT = _E // _NW
_BLK = 128
_NBLKP = 80
_PH = 40
_NP = 10240
_RPT = _NP // _NS

_mesh = plsc.VectorSubcoreMesh(core_axis_name="c", subcore_axis_name="s")


@functools.partial(
    pl.kernel,
    out_type=(jax.ShapeDtypeStruct((_NC, _NP, _H), jnp.float32),
              jax.ShapeDtypeStruct((_NC, _NP), jnp.float32)),
    mesh=_mesh,
    scratch_types=[
        pltpu.VMEM((2 * _PH, _BLK), jnp.int32),
        pltpu.VMEM((_BLK + 1, _H), jnp.float32),
        pltpu.VMEM_SHARED((_NP, _H), jnp.float32),
        pltpu.VMEM_SHARED((_NP,), jnp.float32),
        pltpu.SemaphoreType.DMA,
        pltpu.SemaphoreType.DMA,
    ],
)
def _sc_scatter(x_hbm, att_hbm, src_hbm, dst_hbm, z_hbm, z1_hbm,
                ph_hbm, pa_hbm,
                eidx, rowsb, hacc, aacc,
                semr0, sema0):
    c = lax.axis_index("c")
    s = lax.axis_index("s")
    wid = s * _NC + c

    pltpu.sync_copy(z_hbm.at[pl.ds(s * _RPT, _RPT)],
                    hacc.at[pl.ds(s * _RPT, _RPT)])

    @pl.when(s == 0)
    def _():
        pltpu.sync_copy(z1_hbm, aacc)

    plsc.subcore_barrier()

    rsl = rowsb.at[pl.ds(0, _BLK)]
    asl = rowsb.at[_BLK, pl.ds(0, _BLK)]

    for p in range(2):
        pltpu.sync_copy(src_hbm.at[wid, pl.ds(p * _PH, _PH)],
                        eidx.at[pl.ds(0, _PH)])
        pltpu.sync_copy(dst_hbm.at[wid, pl.ds(p * _PH, _PH)],
                        eidx.at[pl.ds(_PH, _PH)])

        def body(i, carry):
            g = pltpu.async_copy(x_hbm.at[eidx.at[i]], rsl, semr0)
            a = pltpu.async_copy(att_hbm.at[eidx.at[i]], asl, sema0)
            g.wait()
            a.wait()
            pltpu.sync_copy(rsl, hacc.at[eidx.at[_PH + i]], add=True)
            pltpu.sync_copy(asl, aacc.at[eidx.at[_PH + i]], add=True)
            return carry

        lax.fori_loop(0, _PH, body, 0)

    plsc.subcore_barrier()

    pltpu.sync_copy(hacc.at[pl.ds(s * _RPT, _RPT)],
                    ph_hbm.at[c, pl.ds(s * _RPT, _RPT)])

    @pl.when(s == 0)
    def _():
        pltpu.sync_copy(aacc, pa_hbm.at[c])


_BN = 512


def _att_body(x_ref, w_ref, b_ref, o_ref):
    xr = x_ref[...].astype(jnp.bfloat16).astype(jnp.float32)
    wr = w_ref[0:1, :].astype(jnp.bfloat16).astype(jnp.float32)
    att = jnp.sum(xr * wr, axis=1, keepdims=True) + b_ref[0, 0]
    o_ref[...] = jnp.broadcast_to(att, o_ref.shape)


def _combine_body(ph_ref, ab_ref, o_ref):
    h = ph_ref[0] + ph_ref[1]
    o_ref[...] = h / (ab_ref[...] + 1e-8)


def kernel(node_feats, edge_index, W_att, b_att):
    npad = _NW * _NBLKP * _BLK - _E
    src = jnp.pad(edge_index[0], (0, npad)).reshape(_NW, _NBLKP, _BLK)
    dst = jnp.pad(edge_index[1], (0, npad),
                  constant_values=_N).reshape(_NW, _NBLKP, _BLK)
    z = jnp.zeros((_NP, _H), jnp.float32)
    z1 = jnp.zeros((_NP,), jnp.float32)
    wb = jnp.broadcast_to(W_att.reshape(1, _H), (8, _H))
    b2 = b_att.reshape(1, 1)

    attb = pl.pallas_call(
        _att_body,
        grid=(_NP // _BN,),
        in_specs=[
            pl.BlockSpec((_BN, _H), lambda k: (k, 0)),
            pl.BlockSpec((8, _H), lambda k: (0, 0)),
            pl.BlockSpec(memory_space=pltpu.SMEM),
        ],
        out_specs=pl.BlockSpec((_BN, _H), lambda k: (k, 0)),
        out_shape=jax.ShapeDtypeStruct((_NP, _H), jnp.float32),
    )(node_feats, wb, b2)
    att1 = attb[:, 0]

    ph, pa = _sc_scatter(node_feats, att1, src, dst, z, z1)

    asum = pa[0] + pa[1]
    ab = jnp.broadcast_to(asum[:, None], (_NP, _H))
    out = pl.pallas_call(
        _combine_body,
        grid=(_NP // _BN,),
        in_specs=[
            pl.BlockSpec((2, _BN, _H), lambda k: (0, k, 0)),
            pl.BlockSpec((_BN, _H), lambda k: (k, 0)),
        ],
        out_specs=pl.BlockSpec((_BN, _H), lambda k: (k, 0)),
        out_shape=jax.ShapeDtypeStruct((_NP, _H), jnp.float32),
    )(ph, ab)
    return out[:_N]


# speedup vs baseline: 3.1523x; 3.1523x over previous
import functools

import jax
import jax.numpy as jnp
from jax import lax
from jax.experimental import pallas as pl
from jax.experimental.pallas import tpu as pltpu
from jax.experimental.pallas import tpu_sc as plsc

_N = 10000
_H = 128
_E = 320000
_NC = 2
_NS = 16
_NW = _NC * _NS
_EPT = _E // _NW
_BLK = 80
_NBLK = _EPT // _BLK
_PH = 64
_NP = 10240
_RPT = _NP // _NS

_mesh = plsc.VectorSubcoreMesh(core_axis_name="c", subcore_axis_name="s")


@functools.partial(
    pl.kernel,
    out_type=(jax.ShapeDtypeStruct((_NC, _NP, _H), jnp.float32),
              jax.ShapeDtypeStruct((_NC, _NP), jnp.float32)),
    mesh=_mesh,
    scratch_types=[
        pltpu.VMEM((2 * _PH, _BLK), jnp.int32),
        pltpu.VMEM((_BLK, _H), jnp.float32),
        pltpu.VMEM((_BLK, _H), jnp.float32),
        pltpu.VMEM((_BLK,), jnp.float32),
        pltpu.VMEM((_BLK,), jnp.float32),
        pltpu.VMEM_SHARED((_NP, _H), jnp.float32),
        pltpu.VMEM_SHARED((_NP,), jnp.float32),
        pltpu.SemaphoreType.DMA,
        pltpu.SemaphoreType.DMA,
        pltpu.SemaphoreType.DMA,
        pltpu.SemaphoreType.DMA,
    ],
)
def _sc_scatter(x_hbm, att_hbm, src_hbm, dst_hbm, z_hbm, z1_hbm,
                ph_hbm, pa_hbm,
                eidx, rows0, rows1, av0, av1, hacc, aacc,
                semr0, semr1, sema0, sema1):
    c = lax.axis_index("c")
    s = lax.axis_index("s")
    wid = s * _NC + c

    pltpu.sync_copy(z_hbm.at[pl.ds(s * _RPT, _RPT)],
                    hacc.at[pl.ds(s * _RPT, _RPT)])

    @pl.when(s == 0)
    def _():
        pltpu.sync_copy(z1_hbm, aacc)

    plsc.subcore_barrier()

    def start_gather(i, rbuf, abuf, semr, sema):
        pltpu.async_copy(x_hbm.at[eidx.at[i]], rbuf, semr)
        pltpu.async_copy(att_hbm.at[eidx.at[i]], abuf, sema)

    def wait_gather(rbuf, abuf, semr, sema):
        pltpu.make_async_copy(x_hbm.at[eidx.at[0]], rbuf, semr).wait()
        pltpu.make_async_copy(att_hbm.at[eidx.at[0]], abuf, sema).wait()

    def scatter_block(i, rbuf, abuf):
        pltpu.sync_copy(rbuf, hacc.at[eidx.at[_PH + i]], add=True)
        pltpu.sync_copy(abuf, aacc.at[eidx.at[_PH + i]], add=True)

    for p, n in ((0, _PH), (1, _NBLK - _PH)):
        pltpu.sync_copy(src_hbm.at[wid, pl.ds(p * _PH, n)],
                        eidx.at[pl.ds(0, n)])
        pltpu.sync_copy(dst_hbm.at[wid, pl.ds(p * _PH, n)],
                        eidx.at[pl.ds(_PH, n)])
        start_gather(0, rows0, av0, semr0, sema0)

        def pair(j, carry):
            i0 = 2 * j
            start_gather(i0 + 1, rows1, av1, semr1, sema1)
            wait_gather(rows0, av0, semr0, sema0)
            scatter_block(i0, rows0, av0)

            @pl.when(i0 + 2 < n)
            def _():
                start_gather(i0 + 2, rows0, av0, semr0, sema0)

            wait_gather(rows1, av1, semr1, sema1)
            scatter_block(i0 + 1, rows1, av1)
            return carry

        lax.fori_loop(0, n // 2, pair, 0)
        if n % 2:
            wait_gather(rows0, av0, semr0, sema0)
            scatter_block(n - 1, rows0, av0)

    plsc.subcore_barrier()

    pltpu.sync_copy(hacc.at[pl.ds(s * _RPT, _RPT)],
                    ph_hbm.at[c, pl.ds(s * _RPT, _RPT)])

    @pl.when(s == 0)
    def _():
        pltpu.sync_copy(aacc, pa_hbm.at[c])


_BN = 512


def _att_body(x_ref, w_ref, b_ref, o_ref):
    xr = x_ref[...].astype(jnp.bfloat16).astype(jnp.float32)
    wr = w_ref[0:1, :].astype(jnp.bfloat16).astype(jnp.float32)
    att = jnp.sum(xr * wr, axis=1, keepdims=True) + b_ref[0, 0]
    o_ref[...] = jnp.broadcast_to(att, o_ref.shape)


def _combine_body(ph_ref, ab_ref, o_ref):
    h = ph_ref[0] + ph_ref[1]
    o_ref[...] = h / (ab_ref[...] + 1e-8)


def kernel(node_feats, edge_index, W_att, b_att):
    src = edge_index[0].reshape(_NW, _NBLK, _BLK)
    dst = edge_index[1].reshape(_NW, _NBLK, _BLK)
    z = jnp.zeros((_NP, _H), jnp.float32)
    z1 = jnp.zeros((_NP,), jnp.float32)
    wb = jnp.broadcast_to(W_att.reshape(1, _H), (8, _H))
    b2 = b_att.reshape(1, 1)

    attb = pl.pallas_call(
        _att_body,
        grid=(_NP // _BN,),
        in_specs=[
            pl.BlockSpec((_BN, _H), lambda k: (k, 0)),
            pl.BlockSpec((8, _H), lambda k: (0, 0)),
            pl.BlockSpec(memory_space=pltpu.SMEM),
        ],
        out_specs=pl.BlockSpec((_BN, _H), lambda k: (k, 0)),
        out_shape=jax.ShapeDtypeStruct((_NP, _H), jnp.float32),
    )(node_feats, wb, b2)
    att1 = attb[:, 0]

    ph, pa = _sc_scatter(node_feats, att1, src, dst, z, z1)

    asum = pa[0] + pa[1]
    ab = jnp.broadcast_to(asum[:, None], (_NP, _H))
    out = pl.pallas_call(
        _combine_body,
        grid=(_NP // _BN,),
        in_specs=[
            pl.BlockSpec((2, _BN, _H), lambda k: (0, k, 0)),
            pl.BlockSpec((_BN, _H), lambda k: (k, 0)),
        ],
        out_specs=pl.BlockSpec((_BN, _H), lambda k: (k, 0)),
        out_shape=jax.ShapeDtypeStruct((_NP, _H), jnp.float32),
    )(ph, ab)
    return out[:_N]
